# baseline (speedup 1.0000x reference)
import functools

import jax
import jax.numpy as jnp
from jax import lax
from jax.experimental import pallas as pl
from jax.experimental.pallas import tpu as pltpu
from jax.experimental.pallas import tpu_sc as plsc

D = 128
SEQ = 2048
BATCH = 4
B_TOTAL = BATCH * SEQ
NC, NS, L = 2, 16, 16
NW = NC * NS
BPW = B_TOTAL // NW
SCH = SEQ // NW


def _emb_body(idx_hbm, word_hbm, pos_hbm, out_hbm, idx_v, rows_v, pos_v,
              isem, gsem, psem, osem):
    wid = lax.axis_index("s") * NC + lax.axis_index("c")
    s0 = wid * SCH

    idxcp = [
        pltpu.async_copy(idx_hbm.at[pl.ds(b * SEQ + s0, SCH)],
                         idx_v.at[pl.ds(b * SCH, SCH)], isem.at[b])
        for b in range(BATCH)
    ]
    poscp = pltpu.async_copy(pos_hbm.at[pl.ds(s0, SCH)], pos_v, psem)

    for c in idxcp:
        c.wait()
    gather = pltpu.async_copy(word_hbm.at[idx_v], rows_v, gsem)

    poscp.wait()
    gather.wait()
    outs = []
    for b in range(BATCH):

        @plsc.parallel_loop(0, SCH)
        def add_row(j, b=b):
            row = b * SCH + j
            for k in range(D // L):
                sl = pl.ds(k * L, L)
                plsc.addupdate(rows_v.at[row, sl], pos_v[j, sl])
        outs.append(
            pltpu.async_copy(rows_v.at[pl.ds(b * SCH, SCH)],
                             out_hbm.at[pl.ds(b * SEQ + s0, SCH)],
                             osem.at[b]))
    for o in outs:
        o.wait()


@jax.jit
def kernel(inputs, word_embedding, position_embedding):
    idx = inputs.astype(jnp.int32).reshape(B_TOTAL)
    mesh = plsc.VectorSubcoreMesh(core_axis_name="c", subcore_axis_name="s")
    out = pl.kernel(
        _emb_body,
        mesh=mesh,
        out_type=jax.ShapeDtypeStruct((B_TOTAL, D), jnp.float32),
        scratch_types=[
            pltpu.VMEM((BPW,), jnp.int32),
            pltpu.VMEM((BPW, D), jnp.float32),
            pltpu.VMEM((SCH, D), jnp.float32),
            pltpu.SemaphoreType.DMA((BATCH,)),
            pltpu.SemaphoreType.DMA,
            pltpu.SemaphoreType.DMA,
            pltpu.SemaphoreType.DMA((BATCH,)),
        ],
    )(idx, word_embedding, position_embedding)
    return out.reshape(BATCH, SEQ, D)


# speedup vs baseline: 1.0090x; 1.0090x over previous
import functools

import jax
import jax.numpy as jnp
from jax import lax
from jax.experimental import pallas as pl
from jax.experimental.pallas import tpu as pltpu
from jax.experimental.pallas import tpu_sc as plsc

D = 128
SEQ = 2048
BATCH = 4
B_TOTAL = BATCH * SEQ
NC, NS, L = 2, 16, 16
NW = NC * NS
BPW = B_TOTAL // NW
SCH = SEQ // NW


def _emb_body(idx_hbm, word_hbm, pos_hbm, out_hbm, idx_v, rows_v, pos_v,
              isem, gsem, psem, osem):
    wid = lax.axis_index("s") * NC + lax.axis_index("c")
    s0 = wid * SCH

    idxcp = [
        pltpu.async_copy(idx_hbm.at[pl.ds(b * SEQ + s0, SCH)],
                         idx_v.at[b], isem.at[b])
        for b in range(BATCH)
    ]
    poscp = pltpu.async_copy(pos_hbm.at[pl.ds(s0, SCH)], pos_v, psem)

    gathers = []
    for b in range(BATCH):
        idxcp[b].wait()
        gathers.append(
            pltpu.async_copy(word_hbm.at[idx_v.at[b]],
                             rows_v.at[pl.ds(b * SCH, SCH)], gsem.at[b]))

    poscp.wait()
    outs = []
    for b in range(BATCH):
        gathers[b].wait()

        @plsc.parallel_loop(0, SCH)
        def add_row(j, b=b):
            row = b * SCH + j
            for k in range(D // L):
                sl = pl.ds(k * L, L)
                plsc.addupdate(rows_v.at[row, sl], pos_v[j, sl])
        outs.append(
            pltpu.async_copy(rows_v.at[pl.ds(b * SCH, SCH)],
                             out_hbm.at[pl.ds(b * SEQ + s0, SCH)],
                             osem.at[b]))
    for o in outs:
        o.wait()


@jax.jit
def kernel(inputs, word_embedding, position_embedding):
    idx = inputs.astype(jnp.int32).reshape(B_TOTAL)
    mesh = plsc.VectorSubcoreMesh(core_axis_name="c", subcore_axis_name="s")
    out = pl.kernel(
        _emb_body,
        mesh=mesh,
        out_type=jax.ShapeDtypeStruct((B_TOTAL, D), jnp.float32),
        scratch_types=[
            pltpu.VMEM((BATCH, SCH), jnp.int32),
            pltpu.VMEM((BPW, D), jnp.float32),
            pltpu.VMEM((SCH, D), jnp.float32),
            pltpu.SemaphoreType.DMA((BATCH,)),
            pltpu.SemaphoreType.DMA((BATCH,)),
            pltpu.SemaphoreType.DMA,
            pltpu.SemaphoreType.DMA((BATCH,)),
        ],
    )(idx, word_embedding, position_embedding)
    return out.reshape(BATCH, SEQ, D)
